# baseline (speedup 1.0000x reference)
import functools

import jax
import jax.numpy as jnp
from jax import lax
from jax.experimental import pallas as pl
from jax.experimental.pallas import tpu as pltpu
from jax.experimental.pallas import tpu_sc as plsc

ZCH_SIZE = 1000000
HASH_SIZE = 100000
I32MAX = 2**31 - 1

N_VALUES = 819200
NW = 32
PER_W = N_VALUES // NW
CHUNK = 512
NCHUNK = PER_W // CHUNK
NGROUP = CHUNK // 16
SEG = 128
NSEG = CHUNK // SEG
NBUCKET = 62501
LVL1 = 65536

_mesh = plsc.VectorSubcoreMesh(core_axis_name="c", subcore_axis_name="s")


@functools.partial(
    pl.kernel,
    mesh=_mesh,
    compiler_params=pltpu.CompilerParams(
        needs_layout_passes=False, use_tc_tiling_on_sc=False),
    out_type=jax.ShapeDtypeStruct((N_VALUES,), jnp.int32),
    scratch_types=[
        pltpu.VMEM((LVL1,), jnp.int32),
        pltpu.VMEM((16,), jnp.int32),
        pltpu.VMEM((CHUNK,), jnp.int32),
        pltpu.VMEM((CHUNK,), jnp.int32),
        pltpu.VMEM((CHUNK,), jnp.int32),
        pltpu.VMEM((CHUNK,), jnp.int32),
        pltpu.VMEM((CHUNK, 32), jnp.int32),
        pltpu.VMEM((CHUNK, 32), jnp.int32),
        pltpu.VMEM((CHUNK,), jnp.int32),
        pltpu.VMEM((CHUNK,), jnp.int32),
        pltpu.SemaphoreType.DMA,
        pltpu.SemaphoreType.DMA,
    ],
)
def _remap_sc(vals_hbm, lvl1_hbm, comb_hbm, map0_hbm, out_hbm,
              lvl1_v, map0_v, vals0, vals1, bidx0, bidx1,
              rows0, rows1, out0, out1, sem0, sem1):
    i32 = jnp.int32
    cid = lax.axis_index("c")
    sid = lax.axis_index("s")
    wid = sid * i32(2) + cid
    base_w = wid * i32(PER_W)
    pltpu.sync_copy(lvl1_hbm, lvl1_v)
    pltpu.sync_copy(map0_hbm, map0_v)
    iota = lax.iota(jnp.int32, 16)

    def load_bs_fire(c, vals_v, bidx_v, rows_v, sem):
        base = pl.multiple_of(base_w + c * i32(CHUNK), SEG)
        pltpu.sync_copy(vals_hbm.at[pl.ds(base, CHUNK)], vals_v)

        def bs_body(g, carry):
            v = vals_v[pl.ds(g * i32(16), 16)]
            pos = jnp.zeros((16,), jnp.int32)
            for p in (1 << k for k in range(15, -1, -1)):
                probe = plsc.load_gather(lvl1_v, [pos + (p - 1)])
                pos = jnp.where(probe < v, pos + p, pos)
            bidx_v[pl.ds(g * i32(16), 16)] = jnp.maximum(pos - 1, 0)
            return carry

        lax.fori_loop(jnp.int32(0), jnp.int32(NGROUP), bs_body, None)
        for s in range(NSEG):
            pltpu.async_copy(
                comb_hbm.at[bidx_v.at[pl.ds(s * SEG, SEG)]],
                rows_v.at[pl.ds(s * SEG, SEG)],
                sem,
            )

    def wait_rows(bidx_v, rows_v, sem):
        for s in range(NSEG):
            pltpu.make_async_copy(
                comb_hbm.at[bidx_v.at[pl.ds(s * SEG, SEG)]],
                rows_v.at[pl.ds(s * SEG, SEG)],
                sem,
            ).wait()

    def fine_store(c, vals_v, bidx_v, rows_v, out_v):
        map0 = map0_v[pl.ds(0, 16)]

        def fine_body(g, carry):
            v = vals_v[pl.ds(g * i32(16), 16)]
            b = bidx_v[pl.ds(g * i32(16), 16)]
            row_i = g * i32(16) + iota
            cnt = jnp.zeros((16,), jnp.int32)
            eqa = jnp.zeros((16,), jnp.int32)
            for t in range(16):
                col = plsc.load_gather(
                    rows_v, [row_i, jnp.full((16,), t, jnp.int32)])
                cnt = cnt + (col < v).astype(jnp.int32)
                eqa = eqa | (col == v).astype(jnp.int32)
            mapped = plsc.load_gather(rows_v, [row_i, i32(15) + cnt])
            mapped = jnp.where(cnt == i32(0), map0, mapped)
            probe2 = plsc.load_gather(lvl1_v, [b + 1])
            m = jnp.where(cnt == i32(16), (probe2 == v).astype(jnp.int32),
                          eqa)
            h = v % HASH_SIZE + ZCH_SIZE
            out_v[pl.ds(g * i32(16), 16)] = jnp.where(m != i32(0), mapped, h)
            return carry

        lax.fori_loop(jnp.int32(0), jnp.int32(NGROUP), fine_body, None)
        base = pl.multiple_of(base_w + c * i32(CHUNK), SEG)
        pltpu.sync_copy(out_v, out_hbm.at[pl.ds(base, CHUNK)])

    load_bs_fire(i32(0), vals0, bidx0, rows0, sem0)

    def body(k, carry):
        c0 = k * i32(2)
        c1 = c0 + i32(1)
        c2 = lax.rem(c0 + i32(2), i32(NCHUNK))
        load_bs_fire(c1, vals1, bidx1, rows1, sem1)
        wait_rows(bidx0, rows0, sem0)
        fine_store(c0, vals0, bidx0, rows0, out0)
        load_bs_fire(c2, vals0, bidx0, rows0, sem0)
        wait_rows(bidx1, rows1, sem1)
        fine_store(c1, vals1, bidx1, rows1, out1)
        return carry

    lax.fori_loop(jnp.int32(0), jnp.int32(NCHUNK // 2), body, None)
    wait_rows(bidx0, rows0, sem0)


@jax.jit
def kernel(values, mch_sorted_raw_ids, mch_remapped_ids_mapping):
    v32 = values.astype(jnp.int32)
    ids32 = jnp.clip(mch_sorted_raw_ids, 0, I32MAX).astype(jnp.int32)
    ids_pad = jnp.concatenate(
        [ids32, jnp.full((NBUCKET * 16 - ZCH_SIZE - 1,), I32MAX, jnp.int32)])
    buckets = ids_pad.reshape(NBUCKET, 16)
    lvl1 = jnp.concatenate(
        [buckets[:, 0], jnp.full((LVL1 - NBUCKET,), I32MAX, jnp.int32)])
    map32 = mch_remapped_ids_mapping.astype(jnp.int32)
    map_ext = jnp.concatenate([map32, jnp.zeros((32,), jnp.int32)])
    mapshift = lax.dynamic_slice(map_ext, (1,), (NBUCKET * 16,)).reshape(
        NBUCKET, 16)
    comb = jnp.concatenate([buckets, mapshift], axis=1)
    map0 = jnp.broadcast_to(map32[0], (16,))
    out32 = _remap_sc(v32, lvl1, comb, map0)
    return out32.astype(jnp.int64)


# speedup vs baseline: 1.3074x; 1.3074x over previous
import functools

import jax
import jax.numpy as jnp
from jax import lax
from jax.experimental import pallas as pl
from jax.experimental.pallas import tpu as pltpu
from jax.experimental.pallas import tpu_sc as plsc

ZCH_SIZE = 1000000
HASH_SIZE = 100000
I32MAX = 2**31 - 1

N_VALUES = 819200
NW = 32
PER_W = N_VALUES // NW
CHUNK = 512
NCHUNK = PER_W // CHUNK
NGROUP = CHUNK // 16
SEG = 128
NSEG = CHUNK // SEG
NBUCKET = 62501
LVL1 = 65536

_mesh = plsc.VectorSubcoreMesh(core_axis_name="c", subcore_axis_name="s")


@functools.partial(
    pl.kernel,
    mesh=_mesh,
    compiler_params=pltpu.CompilerParams(
        needs_layout_passes=False, use_tc_tiling_on_sc=False),
    out_type=jax.ShapeDtypeStruct((N_VALUES,), jnp.int32),
    scratch_types=[
        pltpu.VMEM((LVL1,), jnp.int32),
        pltpu.VMEM((16,), jnp.int32),
        pltpu.VMEM((PER_W,), jnp.int32),
        pltpu.VMEM((CHUNK,), jnp.int32),
        pltpu.VMEM((CHUNK,), jnp.int32),
        pltpu.VMEM((CHUNK, 32), jnp.int32),
        pltpu.VMEM((CHUNK, 32), jnp.int32),
        pltpu.VMEM((CHUNK,), jnp.int32),
        pltpu.VMEM((CHUNK,), jnp.int32),
        pltpu.SemaphoreType.DMA,
        pltpu.SemaphoreType.DMA,
        pltpu.SemaphoreType.DMA,
        pltpu.SemaphoreType.DMA,
    ],
)
def _remap_sc(vals_hbm, lvl1_hbm, comb_hbm, map0_hbm, out_hbm,
              lvl1_v, map0_v, vals_all, bidx0, bidx1,
              rows0, rows1, out0, out1, sem0, sem1, semo0, semo1):
    i32 = jnp.int32
    cid = lax.axis_index("c")
    sid = lax.axis_index("s")
    wid = sid * i32(2) + cid
    base_w = wid * i32(PER_W)
    pltpu.sync_copy(lvl1_hbm, lvl1_v)
    pltpu.sync_copy(map0_hbm, map0_v)
    pltpu.sync_copy(vals_hbm.at[pl.ds(pl.multiple_of(base_w, SEG), PER_W)],
                    vals_all)
    iota = lax.iota(jnp.int32, 16)

    def load_bs_fire(c, bidx_v, rows_v, sem):
        cbase = c * i32(CHUNK)

        def bs_body(q, carry):
            gbase = q * i32(64)
            vs = [vals_all[pl.ds(cbase + gbase + i32(16 * j), 16)]
                  for j in range(4)]
            poss = [jnp.zeros((16,), jnp.int32) for _ in range(4)]
            for p in (1 << k for k in range(15, -1, -1)):
                for j in range(4):
                    probe = plsc.load_gather(lvl1_v, [poss[j] + (p - 1)])
                    poss[j] = jnp.where(probe < vs[j], poss[j] + p, poss[j])
            for j in range(4):
                bidx_v[pl.ds(gbase + i32(16 * j), 16)] = jnp.maximum(
                    poss[j] - 1, 0)
            return carry

        lax.fori_loop(jnp.int32(0), jnp.int32(NGROUP // 4), bs_body, None)
        for s in range(NSEG):
            pltpu.async_copy(
                comb_hbm.at[bidx_v.at[pl.ds(s * SEG, SEG)]],
                rows_v.at[pl.ds(s * SEG, SEG)],
                sem,
            )

    def wait_rows(bidx_v, rows_v, sem):
        for s in range(NSEG):
            pltpu.make_async_copy(
                comb_hbm.at[bidx_v.at[pl.ds(s * SEG, SEG)]],
                rows_v.at[pl.ds(s * SEG, SEG)],
                sem,
            ).wait()

    def fine_store(c, bidx_v, rows_v, out_v, semo):
        map0 = map0_v[pl.ds(0, 16)]
        cbase = c * i32(CHUNK)

        def fine_body(g, carry):
            v = vals_all[pl.ds(cbase + g * i32(16), 16)]
            b = bidx_v[pl.ds(g * i32(16), 16)]
            row_i = g * i32(16) + iota
            cnt = jnp.zeros((16,), jnp.int32)
            eqa = jnp.zeros((16,), jnp.int32)
            for t in range(16):
                col = plsc.load_gather(
                    rows_v, [row_i, jnp.full((16,), t, jnp.int32)])
                cnt = cnt + (col < v).astype(jnp.int32)
                eqa = eqa | (col == v).astype(jnp.int32)
            mapped = plsc.load_gather(rows_v, [row_i, i32(15) + cnt])
            mapped = jnp.where(cnt == i32(0), map0, mapped)
            probe2 = plsc.load_gather(lvl1_v, [b + 1])
            m = jnp.where(cnt == i32(16), (probe2 == v).astype(jnp.int32),
                          eqa)
            h = v % HASH_SIZE + ZCH_SIZE
            out_v[pl.ds(g * i32(16), 16)] = jnp.where(m != i32(0), mapped, h)
            return carry

        lax.fori_loop(jnp.int32(0), jnp.int32(NGROUP), fine_body, None)
        base = pl.multiple_of(base_w + c * i32(CHUNK), SEG)
        pltpu.async_copy(out_v, out_hbm.at[pl.ds(base, CHUNK)], semo)

    def wait_out(c, out_v, semo):
        base = pl.multiple_of(base_w + c * i32(CHUNK), SEG)
        pltpu.make_async_copy(
            out_v, out_hbm.at[pl.ds(base, CHUNK)], semo).wait()

    pltpu.async_copy(out0, out_hbm.at[pl.ds(base_w, CHUNK)], semo0)
    pltpu.async_copy(
        out1, out_hbm.at[pl.ds(base_w + i32(CHUNK), CHUNK)], semo1)

    load_bs_fire(i32(0), bidx0, rows0, sem0)

    def body(k, carry):
        c0 = k * i32(2)
        c1 = c0 + i32(1)
        c2 = lax.rem(c0 + i32(2), i32(NCHUNK))
        load_bs_fire(c1, bidx1, rows1, sem1)
        wait_rows(bidx0, rows0, sem0)
        wait_out(c0, out0, semo0)
        fine_store(c0, bidx0, rows0, out0, semo0)
        load_bs_fire(c2, bidx0, rows0, sem0)
        wait_rows(bidx1, rows1, sem1)
        wait_out(c1, out1, semo1)
        fine_store(c1, bidx1, rows1, out1, semo1)
        return carry

    lax.fori_loop(jnp.int32(0), jnp.int32(NCHUNK // 2), body, None)
    wait_rows(bidx0, rows0, sem0)
    wait_out(i32(NCHUNK - 2), out0, semo0)
    wait_out(i32(NCHUNK - 1), out1, semo1)


@jax.jit
def kernel(values, mch_sorted_raw_ids, mch_remapped_ids_mapping):
    v32 = values.astype(jnp.int32)
    ids32 = mch_sorted_raw_ids[:ZCH_SIZE].astype(jnp.int32)
    ids_pad = jnp.concatenate(
        [ids32, jnp.full((NBUCKET * 16 - ZCH_SIZE,), I32MAX, jnp.int32)])
    buckets = ids_pad.reshape(NBUCKET, 16)
    lvl1 = jnp.concatenate(
        [buckets[:, 0], jnp.full((LVL1 - NBUCKET,), I32MAX, jnp.int32)])
    map32 = mch_remapped_ids_mapping.astype(jnp.int32)
    map_ext = jnp.concatenate([map32, jnp.zeros((32,), jnp.int32)])
    mapshift = lax.dynamic_slice(map_ext, (1,), (NBUCKET * 16,)).reshape(
        NBUCKET, 16)
    comb = jnp.concatenate([buckets, mapshift], axis=1)
    map0 = jnp.broadcast_to(map32[0], (16,))
    out32 = _remap_sc(v32, lvl1, comb, map0)
    return out32.astype(jnp.int64)
